# baseline (speedup 1.0000x reference)
import functools

import jax
import jax.numpy as jnp
from jax import lax
from jax.experimental import pallas as pl
from jax.experimental.pallas import tpu as pltpu
from jax.experimental.pallas import tpu_sc as plsc

N = 10000
E = 320000
D = 128

NC = 2
NS = 16
NW = NC * NS

C = 100
NB = 3
NGRP = 34
CPW = NGRP * NB
EPAD = NW * CPW * C
NPAIR = NGRP // 2
RPS = (N // NS) // 8 * 8
REM = N - NS * RPS
ZR = 48

_mesh = plsc.VectorSubcoreMesh(core_axis_name="c", subcore_axis_name="s")


@functools.partial(
    pl.kernel,
    out_type=jax.ShapeDtypeStruct((NC, N, D), jnp.float32),
    mesh=_mesh,
    scratch_types=[
        [pltpu.VMEM((NB, C), jnp.int32)] * 2,
        [pltpu.VMEM((NB, C), jnp.int32)] * 2,
        pltpu.VMEM((NB, C, D), jnp.float32),
        pltpu.VMEM((ZR, D), jnp.float32),
        pltpu.VMEM_SHARED((N + 8, D), jnp.float32),
        [pltpu.SemaphoreType.DMA] * NB,
        [pltpu.SemaphoreType.DMA] * NB,
        [pltpu.SemaphoreType.DMA] * 2,
    ],
)
def _sc_agg(h_hbm, src_hbm, dst_hbm, out_hbm, src_ring, dst_ring, rows_v, z_v,
            acc_sh, gsems, ssems, isems):
    cid = lax.axis_index("c")
    sid = lax.axis_index("s")
    wid = cid * NS + sid

    zero = jnp.zeros((16,), jnp.float32)
    for i in range(ZR):
        for j in range(D // 16):
            z_v[i, pl.ds(j * 16, 16)] = zero
    for k in range(RPS // ZR):
        pltpu.sync_copy(z_v, acc_sh.at[pl.ds(sid * RPS + k * ZR, ZR)])

    @pl.when(sid == 0)
    def _():
        pltpu.sync_copy(z_v.at[pl.ds(0, REM)], acc_sh.at[pl.ds(NS * RPS, REM)])

    pltpu.sync_copy(src_hbm.at[wid, 0], src_ring[0])
    pltpu.sync_copy(dst_hbm.at[wid, 0], dst_ring[0])

    plsc.subcore_barrier()

    def _idx_wait(sl):
        pltpu.make_async_copy(src_hbm.at[0, 0], src_ring[sl], isems[sl]).wait()
        pltpu.make_async_copy(src_hbm.at[0, 0], dst_ring[sl], isems[sl]).wait()

    def _idx_prefetch(g, sl):
        pltpu.async_copy(src_hbm.at[wid, g], src_ring[sl], isems[sl])
        pltpu.async_copy(dst_hbm.at[wid, g], dst_ring[sl], isems[sl])

    def _drain_scatter(b):
        pltpu.make_async_copy(h_hbm.at[src_ring[0].at[0]], rows_v.at[b],
                              ssems[b]).wait()

    def _group(g, sl, first_group, prefetch_g, prefetch_cond=None):
        if not first_group:
            _idx_wait(sl)
        handles = []
        for b in range(NB):
            if not first_group:
                _drain_scatter(b)
            handles.append(pltpu.async_copy(
                h_hbm.at[src_ring[sl].at[b]], rows_v.at[b], gsems[b]))
        if prefetch_g is not None:
            if prefetch_cond is None:
                _idx_prefetch(prefetch_g, 1 - sl)
            else:
                @pl.when(prefetch_cond)
                def _():
                    _idx_prefetch(prefetch_g, 1 - sl)
        for b in range(NB):
            handles[b].wait()
            pltpu.async_copy(rows_v.at[b], acc_sh.at[dst_ring[sl].at[b]],
                             ssems[b], add=True)

    _group(0, 0, True, 1)
    _group(1, 1, False, 2)

    def pbody(p, carry):
        _group(2 * p, 0, False, 2 * p + 1)
        _group(2 * p + 1, 1, False, 2 * p + 2, p < NPAIR - 1)
        return carry

    lax.fori_loop(1, NPAIR, pbody, 0)

    for b in range(NB):
        _drain_scatter(b)

    plsc.subcore_barrier()
    pltpu.sync_copy(acc_sh.at[pl.ds(sid * RPS, RPS)],
                    out_hbm.at[cid, pl.ds(sid * RPS, RPS)])

    @pl.when(sid == 0)
    def _():
        pltpu.sync_copy(acc_sh.at[pl.ds(NS * RPS, REM)],
                        out_hbm.at[cid, pl.ds(NS * RPS, REM)])


def _bn(z, g, b):
    m = jnp.mean(z, axis=0, keepdims=True)
    v = jnp.mean((z - m) * (z - m), axis=0, keepdims=True)
    return (z - m) * lax.rsqrt(v + 1e-5) * g + b


def _leaky(z):
    return jnp.where(z >= 0, z, 0.01 * z)


def _mlp_mid_body(h_ref, p_ref, w1_ref, b1_ref, g1_ref, be1_ref, w2_ref,
                  b2_ref, g2_ref, be2_ref, eps_ref, o_ref):
    z = h_ref[...] * eps_ref[...] + p_ref[0] + p_ref[1]
    z = jnp.dot(z, w1_ref[...], preferred_element_type=jnp.float32) + b1_ref[...]
    z = _leaky(_bn(z, g1_ref[...], be1_ref[...]))
    z = jnp.dot(z, w2_ref[...], preferred_element_type=jnp.float32) + b2_ref[...]
    o_ref[...] = _leaky(_bn(z, g2_ref[...], be2_ref[...]))


def _mlp_last_body(h_ref, p_ref, w1_ref, b1_ref, g1_ref, be1_ref, w2_ref,
                   b2_ref, wf_ref, bf_ref, eps_ref, o_ref):
    z = h_ref[...] * eps_ref[...] + p_ref[0] + p_ref[1]
    z = jnp.dot(z, w1_ref[...], preferred_element_type=jnp.float32) + b1_ref[...]
    z = _leaky(_bn(z, g1_ref[...], be1_ref[...]))
    z = jnp.dot(z, w2_ref[...], preferred_element_type=jnp.float32) + b2_ref[...]
    o_ref[...] = jnp.sum(z * wf_ref[...], axis=1, keepdims=True) + bf_ref[...]


_mlp_mid = pl.pallas_call(
    _mlp_mid_body,
    out_shape=jax.ShapeDtypeStruct((N, D), jnp.float32),
)

_mlp_last = pl.pallas_call(
    _mlp_last_body,
    out_shape=jax.ShapeDtypeStruct((N, 1), jnp.float32),
)


def kernel(x, edge_index, W1, b1, g1, be1, W2, b2, eps, g2, be2, Wf, bf):
    pad = EPAD - E
    src = jnp.concatenate(
        [edge_index[0].astype(jnp.int32), jnp.zeros((pad,), jnp.int32)]
    ).reshape(NW, NGRP, NB, C)
    dst = jnp.concatenate(
        [edge_index[1].astype(jnp.int32), jnp.full((pad,), N, jnp.int32)]
    ).reshape(NW, NGRP, NB, C)
    L = W1.shape[0]
    h = x
    for l in range(L):
        parts = _sc_agg(h, src, dst)
        epsb = jnp.full((1, D), 1.0 + eps[l], jnp.float32)
        if l != L - 1:
            h = _mlp_mid(h, parts, W1[l], b1[l].reshape(1, D),
                         g1[l].reshape(1, D), be1[l].reshape(1, D), W2[l],
                         b2[l].reshape(1, D), g2[l].reshape(1, D),
                         be2[l].reshape(1, D), epsb)
        else:
            out = _mlp_last(h, parts, W1[l], b1[l].reshape(1, D),
                            g1[l].reshape(1, D), be1[l].reshape(1, D), W2[l],
                            b2[l].reshape(1, D), Wf.reshape(1, D),
                            bf.reshape(1, 1), epsb)
    return out.reshape(-1)


# speedup vs baseline: 1.6104x; 1.6104x over previous
import functools

import jax
import jax.numpy as jnp
from jax import lax
from jax.experimental import pallas as pl
from jax.experimental.pallas import tpu as pltpu
from jax.experimental.pallas import tpu_sc as plsc

N = 10000
E = 320000
D = 128

NC = 2
NS = 16
NW = NC * NS

C = 80
NB = 3
PH = 3
HCH = 42
CPW = PH * HCH
GPP = HCH // NB
EPAD = NW * CPW * C
RPS = (N // NS) // 8 * 8
REM = N - NS * RPS
ZR = 48

_mesh = plsc.VectorSubcoreMesh(core_axis_name="c", subcore_axis_name="s")


@functools.partial(
    pl.kernel,
    out_type=jax.ShapeDtypeStruct((NC, N, D), jnp.float32),
    mesh=_mesh,
    scratch_types=[
        pltpu.VMEM((HCH, C), jnp.int32),
        pltpu.VMEM((HCH, C), jnp.int32),
        pltpu.VMEM((NB, C, D), jnp.float32),
        pltpu.VMEM((ZR, D), jnp.float32),
        pltpu.VMEM_SHARED((N + 8, D), jnp.float32),
        [pltpu.SemaphoreType.DMA] * NB,
        [pltpu.SemaphoreType.DMA] * NB,
    ],
)
def _sc_agg(h_hbm, src_hbm, dst_hbm, out_hbm, src_v, dst_v, rows_v, z_v,
            acc_sh, gsems, ssems):
    cid = lax.axis_index("c")
    sid = lax.axis_index("s")
    wid = cid * NS + sid

    zero = jnp.zeros((16,), jnp.float32)
    for i in range(ZR):
        for j in range(D // 16):
            z_v[i, pl.ds(j * 16, 16)] = zero
    for k in range(RPS // ZR):
        pltpu.sync_copy(z_v, acc_sh.at[pl.ds(sid * RPS + k * ZR, ZR)])

    @pl.when(sid == 0)
    def _():
        pltpu.sync_copy(z_v.at[pl.ds(0, REM)], acc_sh.at[pl.ds(NS * RPS, REM)])

    plsc.subcore_barrier()

    def _drain_scatter(b):
        pltpu.make_async_copy(h_hbm.at[src_v.at[0]], rows_v.at[b],
                              ssems[b]).wait()

    for ph in range(PH):
        if ph > 0:
            for b in range(NB):
                _drain_scatter(b)
        pltpu.sync_copy(src_hbm.at[wid, ph], src_v)
        pltpu.sync_copy(dst_hbm.at[wid, ph], dst_v)

        def gbody(g, carry):
            handles = []
            for b in range(NB):
                @pl.when(g > 0)
                def _(b=b):
                    _drain_scatter(b)
                handles.append(pltpu.async_copy(
                    h_hbm.at[src_v.at[g * NB + b]], rows_v.at[b], gsems[b]))
            for b in range(NB):
                handles[b].wait()
                pltpu.async_copy(rows_v.at[b], acc_sh.at[dst_v.at[g * NB + b]],
                                 ssems[b], add=True)
            return carry

        lax.fori_loop(0, GPP, gbody, 0)

    for b in range(NB):
        _drain_scatter(b)

    plsc.subcore_barrier()
    pltpu.sync_copy(acc_sh.at[pl.ds(sid * RPS, RPS)],
                    out_hbm.at[cid, pl.ds(sid * RPS, RPS)])

    @pl.when(sid == 0)
    def _():
        pltpu.sync_copy(acc_sh.at[pl.ds(NS * RPS, REM)],
                        out_hbm.at[cid, pl.ds(NS * RPS, REM)])


def _bn(z, g, b):
    m = jnp.mean(z, axis=0, keepdims=True)
    v = jnp.mean((z - m) * (z - m), axis=0, keepdims=True)
    return (z - m) * lax.rsqrt(v + 1e-5) * g + b


def _leaky(z):
    return jnp.where(z >= 0, z, 0.01 * z)


def _mlp_mid_body(h_ref, p_ref, w1_ref, b1_ref, g1_ref, be1_ref, w2_ref,
                  b2_ref, g2_ref, be2_ref, eps_ref, o_ref):
    z = h_ref[...] * eps_ref[...] + p_ref[0] + p_ref[1]
    z = jnp.dot(z, w1_ref[...], preferred_element_type=jnp.float32) + b1_ref[...]
    z = _leaky(_bn(z, g1_ref[...], be1_ref[...]))
    z = jnp.dot(z, w2_ref[...], preferred_element_type=jnp.float32) + b2_ref[...]
    o_ref[...] = _leaky(_bn(z, g2_ref[...], be2_ref[...]))


def _mlp_last_body(h_ref, p_ref, w1_ref, b1_ref, g1_ref, be1_ref, w2_ref,
                   b2_ref, wf_ref, bf_ref, eps_ref, o_ref):
    z = h_ref[...] * eps_ref[...] + p_ref[0] + p_ref[1]
    z = jnp.dot(z, w1_ref[...], preferred_element_type=jnp.float32) + b1_ref[...]
    z = _leaky(_bn(z, g1_ref[...], be1_ref[...]))
    z = jnp.dot(z, w2_ref[...], preferred_element_type=jnp.float32) + b2_ref[...]
    o_ref[...] = jnp.sum(z * wf_ref[...], axis=1, keepdims=True) + bf_ref[...]


_mlp_mid = pl.pallas_call(
    _mlp_mid_body,
    out_shape=jax.ShapeDtypeStruct((N, D), jnp.float32),
)

_mlp_last = pl.pallas_call(
    _mlp_last_body,
    out_shape=jax.ShapeDtypeStruct((N, 1), jnp.float32),
)


def kernel(x, edge_index, W1, b1, g1, be1, W2, b2, eps, g2, be2, Wf, bf):
    pad = EPAD - E
    src = jnp.concatenate(
        [edge_index[0].astype(jnp.int32), jnp.zeros((pad,), jnp.int32)]
    ).reshape(NW, PH, HCH, C)
    dst = jnp.concatenate(
        [edge_index[1].astype(jnp.int32), jnp.full((pad,), N, jnp.int32)]
    ).reshape(NW, PH, HCH, C)
    L = W1.shape[0]
    h = x
    for l in range(L):
        parts = _sc_agg(h, src, dst)
        epsb = jnp.full((1, D), 1.0 + eps[l], jnp.float32)
        if l != L - 1:
            h = _mlp_mid(h, parts, W1[l], b1[l].reshape(1, D),
                         g1[l].reshape(1, D), be1[l].reshape(1, D), W2[l],
                         b2[l].reshape(1, D), g2[l].reshape(1, D),
                         be2[l].reshape(1, D), epsb)
        else:
            out = _mlp_last(h, parts, W1[l], b1[l].reshape(1, D),
                            g1[l].reshape(1, D), be1[l].reshape(1, D), W2[l],
                            b2[l].reshape(1, D), Wf.reshape(1, D),
                            bf.reshape(1, 1), epsb)
    return out.reshape(-1)


# speedup vs baseline: 2.5268x; 1.5691x over previous
import functools

import jax
import jax.numpy as jnp
from jax import lax
from jax.experimental import pallas as pl
from jax.experimental.pallas import tpu as pltpu
from jax.experimental.pallas import tpu_sc as plsc

N = 10000
E = 320000
D = 128

NC = 2
NS = 16
NW = NC * NS

C = 125
NB = 2
PH = 2
HCH = 40
CPW = PH * HCH
GPP = HCH // NB
RPS = (N // NS) // 8 * 8
REM = N - NS * RPS
ZR = 24

_mesh = plsc.VectorSubcoreMesh(core_axis_name="c", subcore_axis_name="s")


@functools.partial(
    pl.kernel,
    out_type=jax.ShapeDtypeStruct((NC, N, D), jnp.float32),
    mesh=_mesh,
    scratch_types=[
        pltpu.VMEM((HCH, C), jnp.int32),
        pltpu.VMEM((HCH, C), jnp.int32),
        pltpu.VMEM((NB, C, D), jnp.float32),
        pltpu.VMEM((ZR, D), jnp.float32),
        pltpu.VMEM_SHARED((N, D), jnp.float32),
        [pltpu.SemaphoreType.DMA] * NB,
        [pltpu.SemaphoreType.DMA] * NB,
    ],
)
def _sc_agg(h_hbm, src_hbm, dst_hbm, out_hbm, src_v, dst_v, rows_v, z_v,
            acc_sh, gsems, ssems):
    cid = lax.axis_index("c")
    sid = lax.axis_index("s")
    wid = cid * NS + sid

    zero = jnp.zeros((16,), jnp.float32)
    for i in range(ZR):
        for j in range(D // 16):
            z_v[i, pl.ds(j * 16, 16)] = zero
    for k in range(RPS // ZR):
        pltpu.sync_copy(z_v, acc_sh.at[pl.ds(sid * RPS + k * ZR, ZR)])

    @pl.when(sid == 0)
    def _():
        pltpu.sync_copy(z_v.at[pl.ds(0, REM)], acc_sh.at[pl.ds(NS * RPS, REM)])

    plsc.subcore_barrier()

    def _drain_scatter(b):
        pltpu.make_async_copy(h_hbm.at[src_v.at[0]], rows_v.at[b],
                              ssems[b]).wait()

    for ph in range(PH):
        if ph > 0:
            for b in range(NB):
                _drain_scatter(b)
        pltpu.sync_copy(src_hbm.at[wid, ph], src_v)
        pltpu.sync_copy(dst_hbm.at[wid, ph], dst_v)

        def gbody(g, carry):
            handles = []
            for b in range(NB):
                @pl.when(g > 0)
                def _(b=b):
                    _drain_scatter(b)
                handles.append(pltpu.async_copy(
                    h_hbm.at[src_v.at[g * NB + b]], rows_v.at[b], gsems[b]))
            for b in range(NB):
                handles[b].wait()
                pltpu.async_copy(rows_v.at[b], acc_sh.at[dst_v.at[g * NB + b]],
                                 ssems[b], add=True)
            return carry

        lax.fori_loop(0, GPP, gbody, 0)

    for b in range(NB):
        _drain_scatter(b)

    plsc.subcore_barrier()
    pltpu.sync_copy(acc_sh.at[pl.ds(sid * RPS, RPS)],
                    out_hbm.at[cid, pl.ds(sid * RPS, RPS)])

    @pl.when(sid == 0)
    def _():
        pltpu.sync_copy(acc_sh.at[pl.ds(NS * RPS, REM)],
                        out_hbm.at[cid, pl.ds(NS * RPS, REM)])


def _bn(z, g, b):
    m = jnp.mean(z, axis=0, keepdims=True)
    v = jnp.mean((z - m) * (z - m), axis=0, keepdims=True)
    return (z - m) * lax.rsqrt(v + 1e-5) * g + b


def _leaky(z):
    return jnp.where(z >= 0, z, 0.01 * z)


LN = 3


def _mlp_mid_body(h_ref, p_ref, w1_ref, b1_ref, g1_ref, be1_ref, w2_ref,
                  b2_ref, g2_ref, be2_ref, eps_ref, o_ref):
    z = h_ref[...] * (1.0 + eps_ref[0]) + p_ref[0] + p_ref[1]
    z = jnp.dot(z, w1_ref[0], preferred_element_type=jnp.float32) + b1_ref[0]
    z = _leaky(_bn(z, g1_ref[0], be1_ref[0]))
    z = jnp.dot(z, w2_ref[0], preferred_element_type=jnp.float32) + b2_ref[0]
    o_ref[...] = _leaky(_bn(z, g2_ref[0], be2_ref[0]))


def _mlp_last_body(h_ref, p_ref, w1_ref, b1_ref, g1_ref, be1_ref, w2_ref,
                   b2_ref, wf_ref, bf_ref, eps_ref, o_ref):
    z = h_ref[...] * (1.0 + eps_ref[0]) + p_ref[0] + p_ref[1]
    z = jnp.dot(z, w1_ref[0], preferred_element_type=jnp.float32) + b1_ref[0]
    z = _leaky(_bn(z, g1_ref[0], be1_ref[0]))
    z = jnp.dot(z, w2_ref[0], preferred_element_type=jnp.float32) + b2_ref[0]
    o_ref[...] = jnp.sum(z * wf_ref[...], axis=1, keepdims=True) + bf_ref[...]


def _wspec(l):
    return pl.BlockSpec((1, D, D), lambda i, l=l: (l, 0, 0))


def _vspec(l):
    return pl.BlockSpec((1, 1, D), lambda i, l=l: (l, 0, 0))


def _sspec(l):
    return pl.BlockSpec((1, 1, 1), lambda i, l=l: (l, 0, 0))


def _make_mid(l):
    return pl.pallas_call(
        _mlp_mid_body,
        out_shape=jax.ShapeDtypeStruct((N, D), jnp.float32),
        in_specs=[
            pl.BlockSpec((N, D), lambda i: (0, 0)),
            pl.BlockSpec((NC, N, D), lambda i: (0, 0, 0)),
            _wspec(l), _vspec(l), _vspec(l), _vspec(l),
            _wspec(l), _vspec(l), _vspec(l), _vspec(l), _sspec(l),
        ],
        out_specs=pl.BlockSpec((N, D), lambda i: (0, 0)),
        grid=(1,),
    )


def _make_last(l):
    return pl.pallas_call(
        _mlp_last_body,
        out_shape=jax.ShapeDtypeStruct((N, 1), jnp.float32),
        in_specs=[
            pl.BlockSpec((N, D), lambda i: (0, 0)),
            pl.BlockSpec((NC, N, D), lambda i: (0, 0, 0)),
            _wspec(l), _vspec(l), _vspec(l), _vspec(l),
            _wspec(l), _vspec(l),
            pl.BlockSpec((1, D), lambda i: (0, 0)),
            pl.BlockSpec((1, 1), lambda i: (0, 0)),
            _sspec(l),
        ],
        out_specs=pl.BlockSpec((N, 1), lambda i: (0, 0)),
        grid=(1,),
    )


_mlp_mid = [_make_mid(l) for l in range(LN - 1)]
_mlp_last = _make_last(LN - 1)


def kernel(x, edge_index, W1, b1, g1, be1, W2, b2, eps, g2, be2, Wf, bf):
    src = edge_index[0].astype(jnp.int32).reshape(NW, PH, HCH, C)
    dst = edge_index[1].astype(jnp.int32).reshape(NW, PH, HCH, C)
    eps2 = eps.reshape(LN, 1, 1)
    b1r, g1r, be1r = (a.reshape(LN, 1, D) for a in (b1, g1, be1))
    b2r, g2r, be2r = (a.reshape(LN, 1, D) for a in (b2, g2, be2))
    wfr = Wf.reshape(1, D)
    bfr = bf.reshape(1, 1)
    h = x
    for l in range(LN):
        parts = _sc_agg(h, src, dst)
        if l != LN - 1:
            h = _mlp_mid[l](h, parts, W1, b1r, g1r, be1r, W2, b2r, g2r,
                            be2r, eps2)
        else:
            out = _mlp_last(h, parts, W1, b1r, g1r, be1r, W2, b2r, wfr,
                            bfr, eps2)
    return out.reshape(-1)
